# Initial kernel scaffold; baseline (speedup 1.0000x reference)
import jax
import jax.numpy as jnp
from jax.experimental import pallas as pl


def kernel(x, edges1, edges2, node2graph, num_relation1, num_relation2):
    raise NotImplementedError("write your pallas kernel here")


# speedup vs baseline: 2.3887x; 2.3887x over previous
import functools

import jax
import jax.numpy as jnp
from jax import lax
from jax.experimental import pallas as pl
from jax.experimental.pallas import tpu as pltpu
from jax.experimental.pallas import tpu_sc as plsc

N = 100000
E1 = 800000
E2 = 800000
E = E1 + E2
B = 64
NW = 32
PER_W = E // NW
CHUNK = 4992
NDMA = CHUNK // 128
NCH = PER_W // CHUNK
TAIL = PER_W - NCH * CHUNK
PER_W1 = E1 // 16
PT_WORDS = N // 4
NODE_W = 25
NODE_PER_W = N // NODE_W

_mesh = plsc.VectorSubcoreMesh(core_axis_name="c", subcore_axis_name="s")
_params = pltpu.CompilerParams(
    needs_layout_passes=False, use_tc_tiling_on_sc=False)

_iota16 = lambda: lax.iota(jnp.int32, 16)


def _worker_id():
    return lax.axis_index("c") * 16 + lax.axis_index("s")


def _edge_slice_start(wid):
    return jnp.where(wid < 16, wid, wid - 16) * PER_W


def _hist_update(hist, key):
    cnt, last = plsc.scan_count(key)
    h = plsc.load_gather(hist, [key])
    plsc.store_scatter(hist, [key], h + cnt, mask=last)


@functools.partial(
    pl.kernel,
    out_type=(
        jax.ShapeDtypeStruct((NW * B,), jnp.int32),
        jax.ShapeDtypeStruct((NW * B,), jnp.int32),
        jax.ShapeDtypeStruct((E,), jnp.int32),
    ),
    mesh=_mesh,
    compiler_params=_params,
    scratch_types=[
        pltpu.VMEM((PT_WORDS,), jnp.int32),
        pltpu.VMEM((CHUNK, 3), jnp.int32),
        pltpu.VMEM((CHUNK,), jnp.int32),
        pltpu.VMEM((B,), jnp.int32),
        pltpu.VMEM((B,), jnp.int32),
        pltpu.VMEM((NODE_PER_W,), jnp.int32),
        pltpu.SemaphoreType.DMA,
    ],
)
def _pass_a(edges1, edges2, ptable_hbm, n2g_hbm,
            hist_e_hbm, hist_n_hbm, keys_hbm,
            ptable, rowbuf, keybuf, hist_e, hist_n, nodebuf, sem):
    wid = _worker_id()
    start = _edge_slice_start(wid)
    pltpu.sync_copy(ptable_hbm, ptable)
    for j in range(B // 16):
        z = jnp.zeros((16,), jnp.int32)
        hist_e[pl.ds(j * 16, 16)] = z
        hist_n[pl.ds(j * 16, 16)] = z

    def do_vec(i, _):
        rowi = _iota16() + i * 16
        src = plsc.load_gather(rowbuf, [rowi, jnp.zeros((16,), jnp.int32)])
        word = plsc.load_gather(ptable, [lax.shift_right_logical(src, 2)])
        key = lax.shift_right_logical(
            word, (src & 3) * 8) & 255
        keybuf[pl.ds(i * 16, 16)] = key
        _hist_update(hist_e, key)
        return 0

    def do_chunk(k, _):
        cstart = start + k * CHUNK

        @pl.when(wid < 16)
        def _():
            pltpu.sync_copy(edges1.at[pl.ds(cstart, CHUNK)], rowbuf)

        @pl.when(wid >= 16)
        def _():
            pltpu.sync_copy(edges2.at[pl.ds(cstart, CHUNK)], rowbuf)

        lax.fori_loop(0, CHUNK // 16, do_vec, 0)
        pltpu.sync_copy(
            keybuf, keys_hbm.at[pl.ds(wid * PER_W + k * CHUNK, CHUNK)])
        return 0

    lax.fori_loop(0, NCH, do_chunk, 0)

    tstart = start + NCH * CHUNK

    @pl.when(wid < 16)
    def _():
        pltpu.sync_copy(edges1.at[pl.ds(tstart, TAIL)],
                        rowbuf.at[pl.ds(0, TAIL)])

    @pl.when(wid >= 16)
    def _():
        pltpu.sync_copy(edges2.at[pl.ds(tstart, TAIL)],
                        rowbuf.at[pl.ds(0, TAIL)])

    lax.fori_loop(0, TAIL // 16, do_vec, 0)
    pltpu.sync_copy(keybuf.at[pl.ds(0, TAIL)],
                    keys_hbm.at[pl.ds(wid * PER_W + NCH * CHUNK, TAIL)])
    pltpu.sync_copy(hist_e, hist_e_hbm.at[pl.ds(wid * B, B)])

    @pl.when(wid < NODE_W)
    def _():
        pltpu.sync_copy(n2g_hbm.at[pl.ds(wid * NODE_PER_W, NODE_PER_W)],
                        nodebuf)

        def do_nvec(i, _):
            _hist_update(hist_n, nodebuf[pl.ds(i * 16, 16)])
            return 0

        lax.fori_loop(0, NODE_PER_W // 16, do_nvec, 0)

    pltpu.sync_copy(hist_n, hist_n_hbm.at[pl.ds(wid * B, B)])


@functools.partial(
    pl.kernel,
    out_type=(
        jax.ShapeDtypeStruct((E, 8), jnp.int32),
        jax.ShapeDtypeStruct((B,), jnp.int32),
    ),
    mesh=_mesh,
    compiler_params=_params,
    scratch_types=[
        pltpu.VMEM((CHUNK, 3), jnp.int32),
        pltpu.VMEM((CHUNK, 8), jnp.int32),
        pltpu.VMEM((CHUNK,), jnp.int32),
        pltpu.VMEM((NDMA, 128), jnp.int32),
        pltpu.VMEM((TAIL,), jnp.int32),
        pltpu.VMEM((NW * B,), jnp.int32),
        pltpu.VMEM((B,), jnp.int32),
        pltpu.VMEM((B,), jnp.int32),
        pltpu.VMEM((B,), jnp.int32),
        pltpu.SemaphoreType.DMA,
    ],
)
def _pass_b(edges1, edges2, keys_hbm, hist_e_hbm, hist_n_hbm, reloff_hbm,
            stage_hbm, nedges_hbm,
            rowbuf, stage, keybuf, pos2d, postail,
            histbuf, base, noff, nedge, sem2):
    wid = _worker_id()
    start = _edge_slice_start(wid)

    pltpu.sync_copy(hist_e_hbm, histbuf)
    for j in range(B // 16):
        acc = jnp.zeros((16,), jnp.int32)
        mine = jnp.zeros((16,), jnp.int32)
        for w in range(NW):
            row = histbuf[pl.ds(w * B + j * 16, 16)]
            acc = acc + row
            wv = jnp.full((16,), w, jnp.int32)
            mine = mine + jnp.where(wv < wid, row, 0)
        nedge[pl.ds(j * 16, 16)] = acc
        base[pl.ds(j * 16, 16)] = mine

    @pl.when(wid == 0)
    def _():
        pltpu.sync_copy(nedge, nedges_hbm)

    carry = jnp.zeros((), jnp.int32)
    for j in range(B // 16):
        v = nedge[pl.ds(j * 16, 16)]
        inc = plsc.cumsum(v)
        base[pl.ds(j * 16, 16)] = base[pl.ds(j * 16, 16)] + inc - v + carry
        carry = carry + jnp.sum(v)

    pltpu.sync_copy(hist_n_hbm, histbuf)
    carry = jnp.zeros((), jnp.int32)
    for j in range(B // 16):
        acc = jnp.zeros((16,), jnp.int32)
        for w in range(NODE_W):
            acc = acc + histbuf[pl.ds(w * B + j * 16, 16)]
        inc = plsc.cumsum(acc)
        noff[pl.ds(j * 16, 16)] = inc - acc + carry
        carry = carry + jnp.sum(acc)

    pltpu.sync_copy(reloff_hbm, keybuf.at[pl.ds(0, 16)])
    reloff = jnp.where(jnp.full((16,), 1, jnp.int32) * wid < 16,
                       jnp.zeros((16,), jnp.int32),
                       keybuf[pl.ds(0, 16)])

    c0 = jnp.zeros((16,), jnp.int32)
    c1 = jnp.full((16,), 1, jnp.int32)
    c2 = jnp.full((16,), 2, jnp.int32)
    c3 = jnp.full((16,), 3, jnp.int32)

    def place_vec(i):
        rowi = _iota16() + i * 16
        key = keybuf[pl.ds(i * 16, 16)]
        cnt, last = plsc.scan_count(key)
        b = plsc.load_gather(base, [key])
        pos = b + cnt - 1
        plsc.store_scatter(base, [key], b + cnt, mask=last)
        s = plsc.load_gather(rowbuf, [rowi, c0])
        d = plsc.load_gather(rowbuf, [rowi, c1])
        r = plsc.load_gather(rowbuf, [rowi, c2])
        off = plsc.load_gather(noff, [key])
        plsc.store_scatter(stage, [rowi, c0], s)
        plsc.store_scatter(stage, [rowi, c1], d)
        plsc.store_scatter(stage, [rowi, c2], r + reloff)
        plsc.store_scatter(stage, [rowi, c3], off)
        return pos

    def do_vec(i, _):
        pos = place_vec(i)
        r = i // 8
        col = (i % 8) * 16
        plsc.store_scatter(pos2d, [jnp.full((16,), r, jnp.int32),
                                   col + _iota16()], pos)
        return 0

    def do_tail_vec(i, _):
        pos = place_vec(i)
        postail[pl.ds(i * 16, 16)] = pos
        return 0

    def do_chunk(k, _):
        cstart = start + k * CHUNK

        @pl.when(wid < 16)
        def _():
            pltpu.sync_copy(edges1.at[pl.ds(cstart, CHUNK)], rowbuf)

        @pl.when(wid >= 16)
        def _():
            pltpu.sync_copy(edges2.at[pl.ds(cstart, CHUNK)], rowbuf)

        pltpu.sync_copy(keys_hbm.at[pl.ds(wid * PER_W + k * CHUNK, CHUNK)],
                        keybuf)
        lax.fori_loop(0, CHUNK // 16, do_vec, 0)
        cps = [pltpu.async_copy(stage.at[pl.ds(j * 128, 128)],
                                stage_hbm.at[pos2d.at[j]], sem2)
               for j in range(NDMA)]
        for cp in cps:
            cp.wait()
        return 0

    lax.fori_loop(0, NCH, do_chunk, 0)

    tstart = start + NCH * CHUNK

    @pl.when(wid < 16)
    def _():
        pltpu.sync_copy(edges1.at[pl.ds(tstart, TAIL)],
                        rowbuf.at[pl.ds(0, TAIL)])

    @pl.when(wid >= 16)
    def _():
        pltpu.sync_copy(edges2.at[pl.ds(tstart, TAIL)],
                        rowbuf.at[pl.ds(0, TAIL)])

    pltpu.sync_copy(keys_hbm.at[pl.ds(wid * PER_W + NCH * CHUNK, TAIL)],
                    keybuf.at[pl.ds(0, TAIL)])
    lax.fori_loop(0, TAIL // 16, do_tail_vec, 0)
    pltpu.async_copy(stage.at[pl.ds(0, TAIL)],
                     stage_hbm.at[postail], sem2).wait()


def kernel(x, edges1, edges2, node2graph, num_relation1, num_relation2):
    ptable = lax.bitcast_convert_type(
        node2graph.astype(jnp.int8).reshape(PT_WORDS, 4), jnp.int32)
    hist_e, hist_n, keys_all = _pass_a(edges1, edges2, ptable, node2graph)
    reloff = jnp.full((16,), jnp.asarray(num_relation1, jnp.int32))
    staging, num_edges = _pass_b(
        edges1, edges2, keys_all, hist_e, hist_n, reloff)
    edge_list = staging[:, :3]
    offsets = staging[:, 3]
    edge_weight = jnp.ones((E,), jnp.float32)
    return (x, edge_list, edge_weight, num_edges, offsets)
